# Initial kernel scaffold; baseline (speedup 1.0000x reference)
import jax
import jax.numpy as jnp
from jax.experimental import pallas as pl


def kernel(in_table, out_table, contexts, targets, negative_sampling):
    raise NotImplementedError("write your pallas kernel here")


# speedup vs baseline: 5.3334x; 5.3334x over previous
import functools

import jax
import jax.numpy as jnp
from jax import lax
from jax.experimental import pallas as pl
from jax.experimental.pallas import tpu as pltpu
from jax.experimental.pallas import tpu_sc as plsc

VOCAB = 1000000
DIM = 64
BATCH = 16384
CTX = 20
NEG = 20

NC = 2
NS = 16
NW = NC * NS
B_PER_W = BATCH // NW
R = 32
NCHUNK = B_PER_W // R
CTX_N = R * CTX
NEG_N = R * NEG
NSUB = CTX_N // 128
FILL = 1.0e9


def _sc_dots(in_table, out_table, ctx2d, tgt_flat, neg2d):
    mesh = plsc.VectorSubcoreMesh(core_axis_name="c", subcore_axis_name="s")

    @functools.partial(
        pl.kernel,
        mesh=mesh,
        out_type=jax.ShapeDtypeStruct((BATCH, 32), jnp.float32),
        compiler_params=pltpu.CompilerParams(
            needs_layout_passes=False, use_tc_tiling_on_sc=False),
        scratch_types=[
            pltpu.VMEM((B_PER_W * CTX // 128, 128), jnp.int32),
            pltpu.VMEM((B_PER_W * NEG // 128, 128), jnp.int32),
            pltpu.VMEM((B_PER_W,), jnp.int32),
            pltpu.VMEM((CTX_N, DIM), jnp.float32),
            pltpu.VMEM((NEG_N, DIM), jnp.float32),
            pltpu.VMEM((R, DIM), jnp.float32),
            pltpu.VMEM((R, 32), jnp.float32),
            pltpu.SemaphoreType.DMA,
        ],
    )
    def k(in_hbm, out_hbm, ctx_hbm, tgt_hbm, neg_hbm, dots_o,
          ctx_idx, neg_idx, tgt_idx, ctx_rows, neg_rows, tgt_rows,
          dots_v, sem):
        wid = lax.axis_index("s") * NC + lax.axis_index("c")
        lane = lax.iota(jnp.int32, 16)
        pltpu.sync_copy(ctx_hbm.at[wid], ctx_idx)
        pltpu.sync_copy(neg_hbm.at[wid], neg_idx)
        pltpu.sync_copy(tgt_hbm.at[pl.ds(wid * B_PER_W, B_PER_W)], tgt_idx)

        def chunk_body(i, _):
            row0 = wid * B_PER_W + i * R
            cps = []
            for s in range(NSUB):
                cps.append(pltpu.async_copy(
                    in_hbm.at[ctx_idx.at[i * NSUB + s]],
                    ctx_rows.at[pl.ds(s * 128, 128)], sem))
                cps.append(pltpu.async_copy(
                    out_hbm.at[neg_idx.at[i * NSUB + s]],
                    neg_rows.at[pl.ds(s * 128, 128)], sem))
            cps.append(pltpu.async_copy(
                out_hbm.at[tgt_idx.at[pl.ds(i * R, R)]], tgt_rows, sem))
            for cp in cps:
                cp.wait()

            def row_body(r, _):
                h = []
                for d in range(DIM // 16):
                    acc = ctx_rows[r * CTX, pl.ds(d * 16, 16)]
                    for c in range(1, CTX):
                        acc = acc + ctx_rows[r * CTX + c, pl.ds(d * 16, 16)]
                    h.append(acc * (1.0 / CTX))
                nh = [-v for v in h]
                acc = tgt_rows[r, pl.ds(0, 16)] * h[0]
                for d in range(1, DIM // 16):
                    acc = acc + tgt_rows[r, pl.ds(d * 16, 16)] * h[d]
                v0 = jnp.where(lane == 0, jnp.sum(acc), jnp.full((16,), FILL))
                v1 = jnp.full((16,), FILL)
                for j in range(NEG):
                    acc = neg_rows[r * NEG + j, pl.ds(0, 16)] * nh[0]
                    for d in range(1, DIM // 16):
                        acc = acc + neg_rows[r * NEG + j, pl.ds(d * 16, 16)] * nh[d]
                    dot = jnp.sum(acc)
                    if j + 1 < 16:
                        v0 = jnp.where(lane == (j + 1), dot, v0)
                    else:
                        v1 = jnp.where(lane == (j + 1 - 16), dot, v1)
                dots_v[r, pl.ds(0, 16)] = v0
                dots_v[r, pl.ds(16, 16)] = v1
                return 0

            lax.fori_loop(0, R, row_body, 0)
            pltpu.sync_copy(dots_v, dots_o.at[pl.ds(row0, R)])
            return 0

        lax.fori_loop(0, NCHUNK, chunk_body, 0)

    return k(in_table, out_table, ctx2d, tgt_flat, neg2d)


def _tc_loss(dots2d):
    def body(dots_ref, out_ref):
        s = -jnp.sum(jax.nn.log_sigmoid(dots_ref[...])) / BATCH
        out_ref[...] = jnp.full((1, 1), s, dtype=jnp.float32)

    out = pl.pallas_call(
        body,
        out_shape=jax.ShapeDtypeStruct((1, 1), jnp.float32),
    )(dots2d)
    return out[0, 0]


def kernel(in_table, out_table, contexts, targets, negative_sampling):
    ctx2d = contexts.astype(jnp.int32).reshape(NW, B_PER_W * CTX // 128, 128)
    neg2d = negative_sampling.astype(jnp.int32).reshape(NW, B_PER_W * NEG // 128, 128)
    tgt_flat = targets.astype(jnp.int32).reshape(BATCH)
    dots = _sc_dots(in_table, out_table, ctx2d, tgt_flat, neg2d)
    return _tc_loss(dots.reshape(BATCH * 32 // 128, 128))
